# baseline (speedup 1.0000x reference)
import jax
import jax.numpy as jnp
from jax.experimental import pallas as pl
from jax.experimental.pallas import tpu as pltpu

K = 1024
D = 64
BETA = 0.25
B = 16
HW = 32 * 32
N_ROWS = B * HW
GRID = B


def _vq_block_kernel(x_ref, e_ref, d_ref, enc_ref, inds_ref, q_ref,
                     loss_ref, perp_ref, counts_ref, acc_ref):
    i = pl.program_id(0)
    xT = x_ref[...]
    e = e_ref[...]

    sx = jnp.sum(xT * xT, axis=0)[:, None]
    se = jnp.sum(e * e, axis=1)
    mm = jax.lax.dot_general(
        xT, e, (((0,), (1,)), ((), ())),
        preferred_element_type=jnp.float32)
    d = (sx + se[None, :]) - 2.0 * mm
    d_ref[...] = d

    m = jnp.min(d, axis=1, keepdims=True)
    iota = jax.lax.broadcasted_iota(jnp.int32, (HW, K), 1)
    masked = jnp.where(d == m, iota, K)
    idx = jnp.min(masked, axis=1, keepdims=True)
    inds_ref[...] = idx

    onehot = (iota == idx).astype(jnp.float32)
    enc_ref[...] = onehot

    qT = jax.lax.dot_general(
        e, onehot, (((0,), (1,)), ((), ())),
        preferred_element_type=jnp.float32)
    q_ref[...] = xT + (qT - xT)

    @pl.when(i == 0)
    def _init():
        counts_ref[...] = jnp.zeros((K,), jnp.float32)
        acc_ref[0] = 0.0

    counts_ref[...] += jnp.sum(onehot, axis=0)
    acc_ref[0] += jnp.sum((qT - xT) ** 2)

    @pl.when(i == GRID - 1)
    def _finish():
        mse = acc_ref[0] / jnp.float32(N_ROWS * D)
        loss_ref[...] = (mse * jnp.float32(BETA) + mse).reshape(1, 1)
        p = counts_ref[...] / jnp.float32(N_ROWS)
        ent = -jnp.sum(p * jnp.log(p + 1e-10))
        perp_ref[...] = jnp.exp(ent).reshape(1, 1)


def kernel(latents, emb_weight):
    lat2 = latents.reshape(B * D, HW)

    d, enc, inds, q, loss, perp = pl.pallas_call(
        _vq_block_kernel,
        grid=(GRID,),
        in_specs=[
            pl.BlockSpec((D, HW), lambda i: (i, 0)),
            pl.BlockSpec((K, D), lambda i: (0, 0)),
        ],
        out_specs=[
            pl.BlockSpec((HW, K), lambda i: (i, 0)),
            pl.BlockSpec((HW, K), lambda i: (i, 0)),
            pl.BlockSpec((HW, 1), lambda i: (i, 0)),
            pl.BlockSpec((D, HW), lambda i: (i, 0)),
            pl.BlockSpec((1, 1), lambda i: (0, 0)),
            pl.BlockSpec((1, 1), lambda i: (0, 0)),
        ],
        out_shape=[
            jax.ShapeDtypeStruct((N_ROWS, K), jnp.float32),
            jax.ShapeDtypeStruct((N_ROWS, K), jnp.float32),
            jax.ShapeDtypeStruct((N_ROWS, 1), jnp.int32),
            jax.ShapeDtypeStruct((B * D, HW), jnp.float32),
            jax.ShapeDtypeStruct((1, 1), jnp.float32),
            jax.ShapeDtypeStruct((1, 1), jnp.float32),
        ],
        scratch_shapes=[
            pltpu.VMEM((K,), jnp.float32),
            pltpu.SMEM((1,), jnp.float32),
        ],
    )(lat2, emb_weight)

    quantize_out = q.reshape(B, D, 32, 32)
    return (quantize_out, loss.reshape(()), perp.reshape(()),
            enc, inds, d)


# speedup vs baseline: 1.0086x; 1.0086x over previous
import jax
import jax.numpy as jnp
from jax.experimental import pallas as pl
from jax.experimental.pallas import tpu as pltpu

K = 1024
D = 64
BETA = 0.25
B = 16
H = 32
W = 32
HW = H * W
N_ROWS = B * HW
GRID = B


def _vq_block_kernel(x_ref, e_ref, d_ref, enc_ref, inds_ref, q_ref,
                     loss_ref, perp_ref, counts_ref, acc_ref):
    i = pl.program_id(0)
    xT = x_ref[...].reshape(D, HW)
    e = e_ref[...]

    sx = jnp.sum(xT * xT, axis=0)[:, None]
    se = jnp.sum(e * e, axis=1)
    mm = jax.lax.dot_general(
        xT, e, (((0,), (1,)), ((), ())),
        preferred_element_type=jnp.float32)
    d = (sx + se[None, :]) - 2.0 * mm
    d_ref[...] = d

    m = jnp.min(d, axis=1, keepdims=True)
    iota = jax.lax.broadcasted_iota(jnp.int32, (HW, K), 1)
    masked = jnp.where(d == m, iota, K)
    idx = jnp.min(masked, axis=1, keepdims=True)
    inds_ref[...] = idx

    onehot = (iota == idx).astype(jnp.float32)
    enc_ref[...] = onehot

    qT = jax.lax.dot_general(
        e, onehot, (((0,), (1,)), ((), ())),
        preferred_element_type=jnp.float32)
    q_ref[...] = (xT + (qT - xT)).reshape(1, D, H, W)

    @pl.when(i == 0)
    def _init():
        counts_ref[...] = jnp.zeros((K,), jnp.float32)
        acc_ref[0] = 0.0

    counts_ref[...] += jnp.sum(onehot, axis=0)
    acc_ref[0] += jnp.sum((qT - xT) ** 2)

    @pl.when(i == GRID - 1)
    def _finish():
        mse = acc_ref[0] / jnp.float32(N_ROWS * D)
        loss_ref[...] = (mse * jnp.float32(BETA) + mse).reshape(1, 1)
        p = counts_ref[...] / jnp.float32(N_ROWS)
        ent = -jnp.sum(p * jnp.log(p + 1e-10))
        perp_ref[...] = jnp.exp(ent).reshape(1, 1)


def kernel(latents, emb_weight):
    d, enc, inds, q, loss, perp = pl.pallas_call(
        _vq_block_kernel,
        grid=(GRID,),
        in_specs=[
            pl.BlockSpec((1, D, H, W), lambda i: (i, 0, 0, 0)),
            pl.BlockSpec((K, D), lambda i: (0, 0)),
        ],
        out_specs=[
            pl.BlockSpec((HW, K), lambda i: (i, 0)),
            pl.BlockSpec((HW, K), lambda i: (i, 0)),
            pl.BlockSpec((HW, 1), lambda i: (i, 0)),
            pl.BlockSpec((1, D, H, W), lambda i: (i, 0, 0, 0)),
            pl.BlockSpec((1, 1), lambda i: (0, 0)),
            pl.BlockSpec((1, 1), lambda i: (0, 0)),
        ],
        out_shape=[
            jax.ShapeDtypeStruct((N_ROWS, K), jnp.float32),
            jax.ShapeDtypeStruct((N_ROWS, K), jnp.float32),
            jax.ShapeDtypeStruct((N_ROWS, 1), jnp.int32),
            jax.ShapeDtypeStruct((B, D, H, W), jnp.float32),
            jax.ShapeDtypeStruct((1, 1), jnp.float32),
            jax.ShapeDtypeStruct((1, 1), jnp.float32),
        ],
        scratch_shapes=[
            pltpu.VMEM((K,), jnp.float32),
            pltpu.SMEM((1,), jnp.float32),
        ],
    )(latents, emb_weight)

    return (q, loss.reshape(()), perp.reshape(()), enc, inds, d)


# speedup vs baseline: 1.5863x; 1.5728x over previous
import functools

import jax
import jax.numpy as jnp
from jax.experimental import pallas as pl
from jax.experimental.pallas import tpu as pltpu

K = 1024
D = 64
BETA = 0.25
N_ROWS = 16 * 32 * 32
BLK = 2048
GRID = N_ROWS // BLK


def _vq_block_kernel(x_ref, e_ref, d_ref, enc_ref, inds_ref, q_ref,
                     loss_ref, perp_ref, counts_ref, acc_ref):
    i = pl.program_id(0)
    x = x_ref[...]
    e = e_ref[...]

    sx = jnp.sum(x * x, axis=1, keepdims=True)
    se = jnp.sum(e * e, axis=1)
    mm = jax.lax.dot_general(
        x, e, (((1,), (1,)), ((), ())),
        preferred_element_type=jnp.float32)
    d = (sx + se[None, :]) - 2.0 * mm
    d_ref[...] = d

    m = jnp.min(d, axis=1, keepdims=True)
    iota = jax.lax.broadcasted_iota(jnp.int32, (BLK, K), 1)
    masked = jnp.where(d == m, iota, K)
    idx = jnp.min(masked, axis=1, keepdims=True)
    inds_ref[...] = idx.reshape(1, 1, BLK)

    onehot = (iota == idx).astype(jnp.float32)
    enc_ref[...] = onehot

    q = jax.lax.dot_general(
        onehot, e, (((1,), (0,)), ((), ())),
        preferred_element_type=jnp.float32)
    q_ref[...] = x + (q - x)

    @pl.when(i == 0)
    def _init():
        counts_ref[...] = jnp.zeros((K,), jnp.float32)
        acc_ref[0] = 0.0

    counts_ref[...] += jnp.sum(onehot, axis=0)
    acc_ref[0] += jnp.sum((q - x) ** 2)

    @pl.when(i == GRID - 1)
    def _finish():
        mse = acc_ref[0] / jnp.float32(N_ROWS * D)
        loss_ref[...] = (mse * jnp.float32(BETA) + mse).reshape(1, 1)
        p = counts_ref[...] / jnp.float32(N_ROWS)
        ent = -jnp.sum(p * jnp.log(p + 1e-10))
        perp_ref[...] = jnp.exp(ent).reshape(1, 1)


@functools.partial(jax.jit, static_argnums=())
def kernel(latents, emb_weight):
    x = jnp.transpose(latents, (0, 2, 3, 1))
    latents_shape = x.shape
    flat = x.reshape(N_ROWS, D)

    d, enc, inds, q, loss, perp = pl.pallas_call(
        _vq_block_kernel,
        grid=(GRID,),
        in_specs=[
            pl.BlockSpec((BLK, D), lambda i: (i, 0)),
            pl.BlockSpec((K, D), lambda i: (0, 0)),
        ],
        out_specs=[
            pl.BlockSpec((BLK, K), lambda i: (i, 0)),
            pl.BlockSpec((BLK, K), lambda i: (i, 0)),
            pl.BlockSpec((1, 1, BLK), lambda i: (i, 0, 0)),
            pl.BlockSpec((BLK, D), lambda i: (i, 0)),
            pl.BlockSpec((1, 1), lambda i: (0, 0)),
            pl.BlockSpec((1, 1), lambda i: (0, 0)),
        ],
        out_shape=[
            jax.ShapeDtypeStruct((N_ROWS, K), jnp.float32),
            jax.ShapeDtypeStruct((N_ROWS, K), jnp.float32),
            jax.ShapeDtypeStruct((GRID, 1, BLK), jnp.int32),
            jax.ShapeDtypeStruct((N_ROWS, D), jnp.float32),
            jax.ShapeDtypeStruct((1, 1), jnp.float32),
            jax.ShapeDtypeStruct((1, 1), jnp.float32),
        ],
        scratch_shapes=[
            pltpu.VMEM((K,), jnp.float32),
            pltpu.SMEM((1,), jnp.float32),
        ],
    )(flat, emb_weight)

    quantize_out = jnp.transpose(q.reshape(latents_shape), (0, 3, 1, 2))
    return (quantize_out, loss.reshape(()), perp.reshape(()),
            enc, inds.reshape(N_ROWS, 1), d)
